# baseline (speedup 1.0000x reference)
import functools

import jax
import jax.numpy as jnp
from jax import lax
from jax.experimental import pallas as pl
from jax.experimental.pallas import tpu as pltpu
from jax.experimental.pallas import tpu_sc as plsc

NUM_RULES = 16384
MEMBERSHIP_DIM = 32

NC = 2
NS = 16
NW = NC * NS
B_PER_W = NUM_RULES // NW
CHUNK = 128
NCHUNK = B_PER_W // CHUNK


def _gather_body(idx_hbm, table_hbm, out_hbm, idx_v, rows_v, sem):
    wid = lax.axis_index("s") * NC + lax.axis_index("c")
    base = wid * B_PER_W
    pltpu.sync_copy(idx_hbm.at[wid], idx_v)
    copies = []
    for j in range(NCHUNK):
        copies.append(
            pltpu.async_copy(
                table_hbm.at[idx_v.at[j]],
                rows_v.at[pl.ds(j * CHUNK, CHUNK)],
                sem,
            )
        )
    for c in copies:
        c.wait()
    pltpu.sync_copy(rows_v, out_hbm.at[pl.ds(base, B_PER_W), 0])


@jax.jit
def _gather(mapping_blocks, table):
    mesh = plsc.VectorSubcoreMesh(core_axis_name="c", subcore_axis_name="s")
    return pl.kernel(
        _gather_body,
        out_type=jax.ShapeDtypeStruct((NUM_RULES, 1, MEMBERSHIP_DIM), jnp.float32),
        mesh=mesh,
        scratch_types=[
            pltpu.VMEM((NCHUNK, CHUNK), jnp.int32),
            pltpu.VMEM((B_PER_W, MEMBERSHIP_DIM), jnp.float32),
            pltpu.SemaphoreType.DMA,
        ],
        compiler_params=pltpu.CompilerParams(use_tc_tiling_on_sc=False),
    )(mapping_blocks, table)


def kernel(x, mapping, table):
    del x
    mapping_blocks = mapping.astype(jnp.int32).reshape(NW, NCHUNK, CHUNK)
    return _gather(mapping_blocks, table)


# speedup vs baseline: 1.2771x; 1.2771x over previous
import functools

import jax
import jax.numpy as jnp
from jax import lax
from jax.experimental import pallas as pl
from jax.experimental.pallas import tpu as pltpu
from jax.experimental.pallas import tpu_sc as plsc

NUM_RULES = 16384
NUM_MEMBERSHIPS = 100000
MEMBERSHIP_DIM = 32

NC = 2
NS = 16
NW = NC * NS
B_PER_W = NUM_RULES // NW
BLK = 128
NBLK = B_PER_W // BLK
L = 16


def _gather_body(map_hbm, tab_hbm, out_hbm, m_v, idx_v, fetched_v, blk_v, sem):
    wid = lax.axis_index("s") * NC + lax.axis_index("c")
    base = wid * B_PER_W

    pltpu.sync_copy(map_hbm.at[pl.ds(base, B_PER_W)], m_v)
    for r in range(B_PER_W // L):
        mm = m_v[pl.ds(r * L, L)]
        idx_v[pl.ds(r * L, L)] = lax.shift_right_logical(mm, 2)

    lanes = lax.iota(jnp.int32, L)

    for db in range(NBLK):
        pltpu.async_copy(
            tab_hbm.at[idx_v.at[pl.ds(db * BLK, BLK)]], fetched_v, sem
        ).wait()

        def extract(jg, _):
            mm = m_v[pl.ds(db * BLK + jg * L, L)]
            off = lax.shift_left(jnp.bitwise_and(mm, 3), 5)
            rows = lanes + jg * L
            for f in range(MEMBERSHIP_DIM):
                val = plsc.load_gather(fetched_v, [rows, off + f])
                blk_v[f // 8, f % 8, pl.ds(jg * L, L)] = val
            return _

        lax.fori_loop(0, BLK // L, extract, None)

        pltpu.sync_copy(
            blk_v, out_hbm.at[:, :, pl.ds(base + db * BLK, BLK)]
        )


@jax.jit
def _gather(mapping, table2):
    mesh = plsc.VectorSubcoreMesh(core_axis_name="c", subcore_axis_name="s")
    return pl.kernel(
        _gather_body,
        out_type=jax.ShapeDtypeStruct((4, 8, NUM_RULES), jnp.float32),
        mesh=mesh,
        scratch_types=[
            pltpu.VMEM((B_PER_W,), jnp.int32),
            pltpu.VMEM((B_PER_W,), jnp.int32),
            pltpu.VMEM((BLK, BLK), jnp.float32),
            pltpu.VMEM((4, 8, BLK), jnp.float32),
            pltpu.SemaphoreType.DMA,
        ],
        compiler_params=pltpu.CompilerParams(
            use_tc_tiling_on_sc=True, needs_layout_passes=False
        ),
    )(mapping, table2)


def kernel(x, mapping, table):
    del x
    table2 = table.reshape(NUM_MEMBERSHIPS // 4, MEMBERSHIP_DIM * 4)
    out3 = _gather(mapping.astype(jnp.int32), table2)
    return out3.reshape(MEMBERSHIP_DIM, NUM_RULES).T.reshape(
        NUM_RULES, 1, MEMBERSHIP_DIM
    )
